# Initial kernel scaffold; baseline (speedup 1.0000x reference)
import jax
import jax.numpy as jnp
from jax.experimental import pallas as pl


def kernel(x, edge_index, W, bias, gamma, beta):
    raise NotImplementedError("write your pallas kernel here")


# speedup vs baseline: 8.5340x; 8.5340x over previous
import functools

import jax
import jax.numpy as jnp
from jax import lax
from jax.experimental import pallas as pl
from jax.experimental.pallas import tpu as pltpu
from jax.experimental.pallas import tpu_sc as plsc


def _make_sc_aggregate(n_nodes, n_edges, n_slices, ch_half):
    NTILES = 16
    CH = 80
    edges_per_tile = n_edges // NTILES
    n_chunks = edges_per_tile // CH
    rows_per_tile = n_nodes // NTILES
    ZR = 128
    n_zero = rows_per_tile // ZR
    assert edges_per_tile * NTILES == n_edges and n_chunks * CH == edges_per_tile
    assert rows_per_tile * NTILES == n_nodes and n_zero * ZR == rows_per_tile
    assert rows_per_tile % 8 == 0

    mesh = plsc.VectorSubcoreMesh(core_axis_name="c", subcore_axis_name="s")

    @functools.partial(
        pl.kernel,
        mesh=mesh,
        out_type=jax.ShapeDtypeStruct((n_slices * 2 * n_nodes, ch_half),
                                      jnp.float32),
        scratch_types=[
            pltpu.VMEM((CH,), jnp.int32),
            pltpu.VMEM((CH,), jnp.int32),
            pltpu.VMEM((CH, ch_half), jnp.float32),
            pltpu.VMEM((ZR, ch_half), jnp.float32),
            pltpu.VMEM_SHARED((n_nodes, ch_half), jnp.float32),
            pltpu.SemaphoreType.DMA,
        ],
    )
    def g_all(u_hbm, rows_hbm, cols_hbm, zeros_hbm, out_hbm,
              idxr, idxc, gbuf, zbuf, acc, sem):
        cid = lax.axis_index("c")
        sid = lax.axis_index("s")
        ebase = sid * edges_per_tile
        rbase = sid * rows_per_tile
        pltpu.sync_copy(zeros_hbm, zbuf)

        def per_slice(s, carry):
            for t in range(n_zero):
                pltpu.sync_copy(zbuf, acc.at[pl.ds(rbase + t * ZR, ZR)])
            plsc.subcore_barrier()

            off = (s * 2 + cid) * n_nodes
            offv = jnp.full((16,), 0, jnp.int32) + off

            def per_chunk(j, carry2):
                base = ebase + j * CH
                pltpu.sync_copy(rows_hbm.at[pl.ds(base, CH)], idxr)
                pltpu.sync_copy(cols_hbm.at[pl.ds(base, CH)], idxc)
                for t in range(CH // 16):
                    sl = pl.ds(t * 16, 16)
                    idxr[sl] = idxr[sl] + offv
                pltpu.async_copy(u_hbm.at[idxr], gbuf, sem).wait()
                pltpu.sync_copy(gbuf, acc.at[idxc], add=True)
                return carry2

            lax.fori_loop(0, n_chunks, per_chunk, 0)
            plsc.subcore_barrier()
            pltpu.sync_copy(acc.at[pl.ds(rbase, rows_per_tile)],
                            out_hbm.at[pl.ds(off + rbase, rows_per_tile)])
            plsc.subcore_barrier()
            return carry

        lax.fori_loop(0, n_slices, per_slice, 0)

    return g_all


def _tc_matmul_body(x_ref, u1_ref, u2_ref, dis_ref, wa_ref, w1_ref, w2_ref,
                    b_ref, o_ref, sum_ref, sq_ref):
    ch = u1_ref.shape[-1]
    x = x_ref[0]
    d = dis_ref[...]
    f32 = jnp.float32
    o = jnp.dot(x, wa_ref[...], preferred_element_type=f32)
    o = o + jnp.dot(u1_ref[0, 0] * (-d), w1_ref[:ch, :],
                    preferred_element_type=f32)
    o = o + jnp.dot(u1_ref[0, 1] * (-d), w1_ref[ch:, :],
                    preferred_element_type=f32)
    o = o + jnp.dot(u2_ref[0, 0] * (2.0 * d), w2_ref[:ch, :],
                    preferred_element_type=f32)
    o = o + jnp.dot(u2_ref[0, 1] * (2.0 * d), w2_ref[ch:, :],
                    preferred_element_type=f32)
    o = o + b_ref[...]
    o_ref[0] = o
    ps = jnp.sum(o, axis=0, keepdims=True)
    pq = jnp.sum(o * o, axis=0, keepdims=True)

    @pl.when(pl.program_id(1) == 0)
    def _init():
        sum_ref[0] = ps
        sq_ref[0] = pq

    @pl.when(pl.program_id(1) != 0)
    def _acc():
        sum_ref[0] += ps
        sq_ref[0] += pq


def _tc_bn_body(o_ref, mu_ref, sc_ref, be_ref, out_ref):
    out_ref[0] = jnp.maximum(
        (o_ref[0] - mu_ref[0]) * sc_ref[0] + be_ref[...], 0.0)


def kernel(x, edge_index, W, bias, gamma, beta):
    B, T, N, C_IN = x.shape
    E = edge_index.shape[1]
    K, _, C_OUT = W.shape
    S = B * T
    CH_HALF = C_IN // 2

    row, col = edge_index[0], edge_index[1]
    deg = jnp.zeros((N,), jnp.float32).at[row].add(1.0)
    dis = jnp.where(deg > 0, lax.rsqrt(deg), 0.0)

    NP = ((N + 2047) // 2048) * 2048
    xf = x.reshape(S, N, C_IN)
    hs = (xf * dis[None, :, None]).reshape(S, N, 2, CH_HALF)
    hs = jnp.pad(hs, ((0, 0), (0, NP - N), (0, 0), (0, 0)))
    hs_flat = hs.transpose(0, 2, 1, 3).reshape(S * 2 * NP, CH_HALF)

    g_all = _make_sc_aggregate(NP, E, S, CH_HALF)
    zeros_tile = jnp.zeros((128, CH_HALF), jnp.float32)

    u1_flat = g_all(hs_flat, row, col, zeros_tile)
    d2p = jnp.pad(dis * dis, (0, NP - N))
    d2t = jnp.tile(d2p, S * 2)[:, None]
    u2_flat = g_all(u1_flat * d2t, row, col, zeros_tile)

    u1r = u1_flat.reshape(S, 2, NP, CH_HALF)[:, :, :N, :]
    u2r = u2_flat.reshape(S, 2, NP, CH_HALF)[:, :, :N, :]

    NB = 5
    BN_ROWS = N // NB
    o, sums, sq = pl.pallas_call(
        _tc_matmul_body,
        grid=(S, NB),
        in_specs=[
            pl.BlockSpec((1, BN_ROWS, C_IN), lambda s, b: (s, b, 0)),
            pl.BlockSpec((1, 2, BN_ROWS, CH_HALF), lambda s, b: (s, 0, b, 0)),
            pl.BlockSpec((1, 2, BN_ROWS, CH_HALF), lambda s, b: (s, 0, b, 0)),
            pl.BlockSpec((BN_ROWS, 1), lambda s, b: (b, 0)),
            pl.BlockSpec((C_IN, C_OUT), lambda s, b: (0, 0)),
            pl.BlockSpec((C_IN, C_OUT), lambda s, b: (0, 0)),
            pl.BlockSpec((C_IN, C_OUT), lambda s, b: (0, 0)),
            pl.BlockSpec((1, C_OUT), lambda s, b: (0, 0)),
        ],
        out_specs=[
            pl.BlockSpec((1, BN_ROWS, C_OUT), lambda s, b: (s, b, 0)),
            pl.BlockSpec((1, 1, C_OUT), lambda s, b: (s, 0, 0)),
            pl.BlockSpec((1, 1, C_OUT), lambda s, b: (s, 0, 0)),
        ],
        out_shape=[
            jax.ShapeDtypeStruct((S, N, C_OUT), jnp.float32),
            jax.ShapeDtypeStruct((S, 1, C_OUT), jnp.float32),
            jax.ShapeDtypeStruct((S, 1, C_OUT), jnp.float32),
        ],
    )(xf, u1r, u2r, dis[:, None], W[0] - W[2], W[1], W[2], bias[None, :])

    mu = sums / N
    var = sq / N - mu * mu
    scale = gamma[None, None, :] * lax.rsqrt(var + 1e-5)

    out = pl.pallas_call(
        _tc_bn_body,
        grid=(S, NB),
        in_specs=[
            pl.BlockSpec((1, BN_ROWS, C_OUT), lambda s, b: (s, b, 0)),
            pl.BlockSpec((1, 1, C_OUT), lambda s, b: (s, 0, 0)),
            pl.BlockSpec((1, 1, C_OUT), lambda s, b: (s, 0, 0)),
            pl.BlockSpec((1, C_OUT), lambda s, b: (0, 0)),
        ],
        out_specs=pl.BlockSpec((1, BN_ROWS, C_OUT), lambda s, b: (s, b, 0)),
        out_shape=jax.ShapeDtypeStruct((S, N, C_OUT), jnp.float32),
    )(o, mu, scale, beta[None, :])

    return out.reshape(B, T, N, C_OUT)


# speedup vs baseline: 13.3479x; 1.5641x over previous
import functools

import jax
import jax.numpy as jnp
from jax import lax
from jax.experimental import pallas as pl
from jax.experimental.pallas import tpu as pltpu
from jax.experimental.pallas import tpu_sc as plsc


def _make_sc_aggregate(n_nodes, n_edges, n_slices, ch_half):
    NTILES = 16
    CH = 80
    edges_per_tile = n_edges // NTILES
    n_chunks = edges_per_tile // CH
    rows_per_tile = n_nodes // NTILES
    ZR = 128
    n_zero = rows_per_tile // ZR
    assert edges_per_tile * NTILES == n_edges and n_chunks * CH == edges_per_tile
    assert rows_per_tile * NTILES == n_nodes and n_zero * ZR == rows_per_tile
    assert rows_per_tile % 8 == 0

    assert n_chunks % 2 == 1
    mesh = plsc.VectorSubcoreMesh(core_axis_name="c", subcore_axis_name="s")

    @functools.partial(
        pl.kernel,
        mesh=mesh,
        out_type=jax.ShapeDtypeStruct((n_slices * 2 * n_nodes, ch_half),
                                      jnp.float32),
        scratch_types=[
            pltpu.VMEM((CH,), jnp.int32),
            pltpu.VMEM((CH,), jnp.int32),
            pltpu.VMEM((CH,), jnp.int32),
            pltpu.VMEM((CH,), jnp.int32),
            pltpu.VMEM((CH, ch_half), jnp.float32),
            pltpu.VMEM((CH, ch_half), jnp.float32),
            pltpu.VMEM((ZR, ch_half), jnp.float32),
            pltpu.VMEM_SHARED((n_nodes, ch_half), jnp.float32),
            pltpu.SemaphoreType.DMA,
            pltpu.SemaphoreType.DMA,
        ],
    )
    def g_all(u_hbm, rows_hbm, cols_hbm, zeros_hbm, out_hbm,
              idx0, idx1, idxc0, idxc1, gbuf0, gbuf1, zbuf, acc,
              sem0, sem1):
        cid = lax.axis_index("c")
        sid = lax.axis_index("s")
        ebase = sid * edges_per_tile
        rbase = sid * rows_per_tile
        pltpu.sync_copy(zeros_hbm, zbuf)

        def per_slice(s, carry):
            for t in range(n_zero):
                pltpu.sync_copy(zbuf, acc.at[pl.ds(rbase + t * ZR, ZR)])
            plsc.subcore_barrier()

            off = (s * 2 + cid) * n_nodes
            offv = jnp.full((16,), 0, jnp.int32) + off

            def prep(idx, idxc, c):
                base = ebase + c * CH
                pltpu.sync_copy(rows_hbm.at[pl.ds(base, CH)], idx)
                pltpu.sync_copy(cols_hbm.at[pl.ds(base, CH)], idxc)
                for t in range(CH // 16):
                    sl = pl.ds(t * 16, 16)
                    idx[sl] = idx[sl] + offv

            def gather(idx, gbuf, sem):
                pltpu.async_copy(u_hbm.at[idx], gbuf, sem)

            def wait(idx, gbuf, sem):
                pltpu.make_async_copy(u_hbm.at[idx], gbuf, sem).wait()

            def scat(gbuf, idxc):
                pltpu.sync_copy(gbuf, acc.at[idxc], add=True)

            prep(idx0, idxc0, 0)
            gather(idx0, gbuf0, sem0)

            def per_pair(j, carry2):
                c0 = 2 * j
                prep(idx1, idxc1, c0 + 1)
                gather(idx1, gbuf1, sem1)
                wait(idx0, gbuf0, sem0)
                scat(gbuf0, idxc0)
                prep(idx0, idxc0, c0 + 2)
                gather(idx0, gbuf0, sem0)
                wait(idx1, gbuf1, sem1)
                scat(gbuf1, idxc1)
                return carry2

            lax.fori_loop(0, (n_chunks - 1) // 2, per_pair, 0)
            wait(idx0, gbuf0, sem0)
            scat(gbuf0, idxc0)

            plsc.subcore_barrier()
            pltpu.sync_copy(acc.at[pl.ds(rbase, rows_per_tile)],
                            out_hbm.at[pl.ds(off + rbase, rows_per_tile)])
            plsc.subcore_barrier()
            return carry

        lax.fori_loop(0, n_slices, per_slice, 0)

    return g_all


def _tc_matmul_body(x_ref, u1_ref, u2_ref, dis_ref, wa_ref, w1_ref, w2_ref,
                    b_ref, o_ref, sum_ref, sq_ref):
    ch = u1_ref.shape[-1]
    x = x_ref[0]
    d = dis_ref[...]
    f32 = jnp.float32
    o = jnp.dot(x, wa_ref[...], preferred_element_type=f32)
    o = o + jnp.dot(u1_ref[0, 0] * (-d), w1_ref[:ch, :],
                    preferred_element_type=f32)
    o = o + jnp.dot(u1_ref[0, 1] * (-d), w1_ref[ch:, :],
                    preferred_element_type=f32)
    o = o + jnp.dot(u2_ref[0, 0] * (2.0 * d), w2_ref[:ch, :],
                    preferred_element_type=f32)
    o = o + jnp.dot(u2_ref[0, 1] * (2.0 * d), w2_ref[ch:, :],
                    preferred_element_type=f32)
    o = o + b_ref[...]
    o_ref[0] = o
    ps = jnp.sum(o, axis=0, keepdims=True)
    pq = jnp.sum(o * o, axis=0, keepdims=True)

    @pl.when(pl.program_id(1) == 0)
    def _init():
        sum_ref[0] = ps
        sq_ref[0] = pq

    @pl.when(pl.program_id(1) != 0)
    def _acc():
        sum_ref[0] += ps
        sq_ref[0] += pq


def _tc_bn_body(o_ref, mu_ref, sc_ref, be_ref, out_ref):
    out_ref[0] = jnp.maximum(
        (o_ref[0] - mu_ref[0]) * sc_ref[0] + be_ref[...], 0.0)


def kernel(x, edge_index, W, bias, gamma, beta):
    B, T, N, C_IN = x.shape
    E = edge_index.shape[1]
    K, _, C_OUT = W.shape
    S = B * T
    CH_HALF = C_IN // 2

    row, col = edge_index[0], edge_index[1]
    deg = jnp.zeros((N,), jnp.float32).at[row].add(1.0)
    dis = jnp.where(deg > 0, lax.rsqrt(deg), 0.0)

    NP = ((N + 2047) // 2048) * 2048
    xf = x.reshape(S, N, C_IN)
    hs = (xf * dis[None, :, None]).reshape(S, N, 2, CH_HALF)
    hs = jnp.pad(hs, ((0, 0), (0, NP - N), (0, 0), (0, 0)))
    hs_flat = hs.transpose(0, 2, 1, 3).reshape(S * 2 * NP, CH_HALF)

    g_all = _make_sc_aggregate(NP, E, S, CH_HALF)
    zeros_tile = jnp.zeros((128, CH_HALF), jnp.float32)

    u1_flat = g_all(hs_flat, row, col, zeros_tile)
    d2p = jnp.pad(dis * dis, (0, NP - N))
    d2t = jnp.tile(d2p, S * 2)[:, None]
    u2_flat = g_all(u1_flat * d2t, row, col, zeros_tile)

    u1r = u1_flat.reshape(S, 2, NP, CH_HALF)[:, :, :N, :]
    u2r = u2_flat.reshape(S, 2, NP, CH_HALF)[:, :, :N, :]

    NB = 5
    BN_ROWS = N // NB
    o, sums, sq = pl.pallas_call(
        _tc_matmul_body,
        grid=(S, NB),
        in_specs=[
            pl.BlockSpec((1, BN_ROWS, C_IN), lambda s, b: (s, b, 0)),
            pl.BlockSpec((1, 2, BN_ROWS, CH_HALF), lambda s, b: (s, 0, b, 0)),
            pl.BlockSpec((1, 2, BN_ROWS, CH_HALF), lambda s, b: (s, 0, b, 0)),
            pl.BlockSpec((BN_ROWS, 1), lambda s, b: (b, 0)),
            pl.BlockSpec((C_IN, C_OUT), lambda s, b: (0, 0)),
            pl.BlockSpec((C_IN, C_OUT), lambda s, b: (0, 0)),
            pl.BlockSpec((C_IN, C_OUT), lambda s, b: (0, 0)),
            pl.BlockSpec((1, C_OUT), lambda s, b: (0, 0)),
        ],
        out_specs=[
            pl.BlockSpec((1, BN_ROWS, C_OUT), lambda s, b: (s, b, 0)),
            pl.BlockSpec((1, 1, C_OUT), lambda s, b: (s, 0, 0)),
            pl.BlockSpec((1, 1, C_OUT), lambda s, b: (s, 0, 0)),
        ],
        out_shape=[
            jax.ShapeDtypeStruct((S, N, C_OUT), jnp.float32),
            jax.ShapeDtypeStruct((S, 1, C_OUT), jnp.float32),
            jax.ShapeDtypeStruct((S, 1, C_OUT), jnp.float32),
        ],
    )(xf, u1r, u2r, dis[:, None], W[0] - W[2], W[1], W[2], bias[None, :])

    mu = sums / N
    var = sq / N - mu * mu
    scale = gamma[None, None, :] * lax.rsqrt(var + 1e-5)

    out = pl.pallas_call(
        _tc_bn_body,
        grid=(S, NB),
        in_specs=[
            pl.BlockSpec((1, BN_ROWS, C_OUT), lambda s, b: (s, b, 0)),
            pl.BlockSpec((1, 1, C_OUT), lambda s, b: (s, 0, 0)),
            pl.BlockSpec((1, 1, C_OUT), lambda s, b: (s, 0, 0)),
            pl.BlockSpec((1, C_OUT), lambda s, b: (0, 0)),
        ],
        out_specs=pl.BlockSpec((1, BN_ROWS, C_OUT), lambda s, b: (s, b, 0)),
        out_shape=jax.ShapeDtypeStruct((S, N, C_OUT), jnp.float32),
    )(o, mu, scale, beta[None, :])

    return out.reshape(B, T, N, C_OUT)
